# baseline (speedup 1.0000x reference)
import functools

import numpy as np

import jax
import jax.numpy as jnp
from jax import lax
from jax.experimental import pallas as pl
from jax.experimental.pallas import tpu as pltpu
from jax.experimental.pallas import tpu_sc as plsc

_N = 4096
_NB = 64
_NW = 32
_RPW = _N // _NW
_IV = _RPW // 16
_CLAMP_HI = float(np.nextafter(np.float32(16.0), np.float32(0.0)))


def _sc_occupancy(xi, yi, rowdest, xj, yj, lo_c, hi_c):
    mesh = plsc.VectorSubcoreMesh(core_axis_name="c", subcore_axis_name="s")

    @functools.partial(
        pl.kernel,
        out_type=jax.ShapeDtypeStruct((_N * _NB,), jnp.float32),
        mesh=mesh,
        scratch_types=[
            pltpu.VMEM((_N,), jnp.float32),
            pltpu.VMEM((_N,), jnp.float32),
            pltpu.VMEM((_RPW,), jnp.float32),
            pltpu.VMEM((_RPW,), jnp.float32),
            pltpu.VMEM((_RPW,), jnp.int32),
            pltpu.VMEM((16,), jnp.int32),
            pltpu.VMEM((16,), jnp.int32),
            pltpu.VMEM((_RPW * _NB,), jnp.float32),
        ],
        compiler_params=pltpu.CompilerParams(needs_layout_passes=False),
    )
    def occ_kernel(xi_hbm, yi_hbm, rd_hbm, xj_hbm, yj_hbm, lo_hbm, hi_hbm,
                   occ_hbm, xj_v, yj_v, xi_v, yi_v, rd_v, lo_v, hi_v, occ_v):
        cid = lax.axis_index("c")
        sid = lax.axis_index("s")
        wid = sid * 2 + cid
        base = pl.multiple_of(wid * _RPW, _RPW)
        vbase = pl.multiple_of(wid * _IV, _IV)

        pltpu.sync_copy(xj_hbm, xj_v)
        pltpu.sync_copy(yj_hbm, yj_v)
        pltpu.sync_copy(xi_hbm.at[pl.ds(base, _RPW)], xi_v)
        pltpu.sync_copy(yi_hbm.at[pl.ds(base, _RPW)], yi_v)
        pltpu.sync_copy(rd_hbm.at[pl.ds(base, _RPW)], rd_v)
        pltpu.sync_copy(lo_hbm.at[pl.ds(vbase, 16)], lo_v)
        pltpu.sync_copy(hi_hbm.at[pl.ds(vbase, 16)], hi_v)

        zero16 = jnp.zeros((16,), jnp.float32)

        def zbody(k, carry):
            occ_v[pl.ds(k * 16, 16)] = zero16
            return carry

        lax.fori_loop(0, _RPW * _NB // 16, zbody, 0)

        ones = jnp.ones((16,), jnp.float32)
        los = lo_v[pl.ds(0, 16)]
        his = hi_v[pl.ds(0, 16)]

        for k in range(_IV):
            k0 = pl.multiple_of(k * 16, 16)
            cx = xi_v[pl.ds(k0, 16)] - 12.0
            cy = yi_v[pl.ds(k0, 16)] - 12.0
            rb = rd_v[pl.ds(k0, 16)] * _NB - 0x410

            def jbody(jc, carry, cx=cx, cy=cy, rb=rb):
                j0 = pl.multiple_of(jc * 16, 16)
                xchunk = xj_v[pl.ds(j0, 16)]
                ychunk = yj_v[pl.ds(j0, 16)]
                for jj in range(16):
                    xjb = jnp.full((16,), xchunk[jj], jnp.float32)
                    yjb = jnp.full((16,), ychunk[jj], jnp.float32)
                    rx = xjb - cx
                    ry = yjb - cy
                    rx = jnp.minimum(jnp.maximum(rx, 8.0), _CLAMP_HI)
                    ry = jnp.minimum(jnp.maximum(ry, 8.0), _CLAMP_HI)
                    bxx = plsc.bitcast(rx, jnp.int32)
                    byy = plsc.bitcast(ry, jnp.int32)
                    col = lax.shift_right_logical(bxx, 17) & 0x38
                    idx = (rb + col) + lax.shift_right_logical(byy, 20)
                    plsc.addupdate_scatter(occ_v, [idx], ones)
                return carry

            lax.fori_loop(los[k], his[k], jbody, 0)

        pltpu.sync_copy(
            occ_v, occ_hbm.at[pl.ds(pl.multiple_of(base * _NB, 8), _RPW * _NB)]
        )

    return occ_kernel(xi, yi, rowdest, xj, yj, lo_c, hi_c)


def _tc_linear(occ64, w64, b2):
    def mm_kernel(occ_ref, w_ref, b_ref, o_ref):
        o_ref[...] = (
            jnp.dot(occ_ref[...], w_ref[...], preferred_element_type=jnp.float32)
            + b_ref[...]
        )

    return pl.pallas_call(
        mm_kernel,
        grid=(8,),
        in_specs=[
            pl.BlockSpec((_N // 8, _NB), lambda i: (i, 0)),
            pl.BlockSpec((_NB, 128), lambda i: (0, 0)),
            pl.BlockSpec((1, 128), lambda i: (0, 0)),
        ],
        out_specs=pl.BlockSpec((_N // 8, 128), lambda i: (i, 0)),
        out_shape=jax.ShapeDtypeStruct((_N, 128), jnp.float32),
    )(occ64, w64, b2)


@jax.jit
def kernel(hidden_in, cell_in, obs, W, b):
    del hidden_in, cell_in
    obs_x = obs[:, 0]
    obs_y = obs[:, 1]

    y_blocks = obs_y.reshape(_NW, _RPW)
    ordl = jnp.argsort(y_blocks, axis=1)
    gidx = (ordl + _RPW * jnp.arange(_NW, dtype=ordl.dtype)[:, None]).reshape(-1)
    xi = obs_x[gidx] * 2.0
    yi = obs_y[gidx] * 2.0
    rowdest = ordl.reshape(-1).astype(jnp.int32)

    gord = jnp.argsort(obs_y)
    xj = obs_x[gord] * 2.0
    yj = obs_y[gord] * 2.0

    yiv = yi.reshape(_N // 16, 16)
    ymin = yiv.min(axis=1)
    ymax = yiv.max(axis=1)
    lo = jnp.searchsorted(yj, ymin - 3.001)
    hi = jnp.searchsorted(yj, ymax + 3.001, side="right")
    lo_c = jnp.zeros((512,), jnp.int32).at[: _N // 16].set(
        (lo // 16).astype(jnp.int32))
    hi_c = jnp.zeros((512,), jnp.int32).at[: _N // 16].set(
        ((hi + 15) // 16).astype(jnp.int32))

    occ64 = _sc_occupancy(xi, yi, rowdest, xj, yj, lo_c, hi_c).reshape(_N, _NB)

    c36 = jnp.arange(36, dtype=jnp.int32)
    cols = (c36 // 6) * 8 + (c36 % 6) + 9
    w64 = jnp.zeros((_NB, 128), jnp.float32).at[cols].set(W.T)
    b2 = (b - W[:, 21])[None, :]

    return _tc_linear(occ64, w64, b2)
